# Initial kernel scaffold; baseline (speedup 1.0000x reference)
import jax
import jax.numpy as jnp
from jax.experimental import pallas as pl


def kernel(x, edge_index, edge_attr, params):
    raise NotImplementedError("write your pallas kernel here")


# speedup vs baseline: 2.3649x; 2.3649x over previous
import functools

import jax
import jax.numpy as jnp
from jax import lax
from jax.experimental import pallas as pl
from jax.experimental.pallas import tpu as pltpu
from jax.experimental.pallas import tpu_sc as plsc

NC = 2
NS = 16
NW = NC * NS
BLK = 128
TW = 48
FAT = 128
EOP = 16


def _pad_cols(w, k):
    return jnp.pad(w, ((0, 0), (0, k - w.shape[1])))


def _pad_rows(w, k):
    return jnp.pad(w, ((0, k - w.shape[0]), (0, 0)))


def _tc_prep(cur, wab):
    n = cur.shape[0]

    def body(c_ref, w_ref, t_ref):
        t_ref[...] = jnp.dot(c_ref[...], w_ref[...],
                             preferred_element_type=jnp.float32)

    return pl.pallas_call(
        body,
        out_shape=jax.ShapeDtypeStruct((n, FAT), jnp.float32),
    )(cur, wab)


def _tc_edge_mlp(s, ea, wc, b1, w2, b2, w3, b3, eout_true):
    e = s.shape[0]
    blk = 8000
    grid = e // blk
    ein = ea.shape[1]
    outs = [jax.ShapeDtypeStruct((e, EOP), jnp.float32)]
    out_specs = [pl.BlockSpec((blk, EOP), lambda i: (i, 0))]
    if eout_true != EOP:
        outs.append(jax.ShapeDtypeStruct((e, eout_true), jnp.float32))
        out_specs.append(pl.BlockSpec((blk, eout_true), lambda i: (i, 0)))

    def body(s_ref, ea_ref, wc_ref, b1_ref, w2_ref, b2_ref, w3_ref,
             b3_ref, mp_ref, *rest):
        h = (jnp.dot(ea_ref[...], wc_ref[...], preferred_element_type=jnp.float32)
             + s_ref[...] + b1_ref[...])
        h = jnp.maximum(h, 0.0)
        h = jnp.maximum(
            jnp.dot(h, w2_ref[...], preferred_element_type=jnp.float32) + b2_ref[...],
            0.0)
        mp = jnp.dot(h, w3_ref[...], preferred_element_type=jnp.float32) + b3_ref[...]
        mp_ref[...] = mp
        if rest:
            rest[0][...] = mp[:, :eout_true]

    full = lambda a: pl.BlockSpec(a.shape, lambda i: tuple(0 for _ in a.shape))
    return pl.pallas_call(
        body,
        grid=(grid,),
        in_specs=[
            pl.BlockSpec((blk, TW), lambda i: (i, 0)),
            pl.BlockSpec((blk, ein), lambda i: (i, 0)),
            full(wc), full(b1), full(w2), full(b2), full(w3), full(b3),
        ],
        out_specs=out_specs,
        out_shape=outs,
        compiler_params=pltpu.CompilerParams(
            dimension_semantics=("arbitrary",)),
    )(s, ea, wc, b1, w2, b2, w3, b3)


def _tc_node_update(cur, part, va, vbp, c1, v2, c2, v3, c3, enc, nwa):
    n, nout = cur.shape[0], v3.shape[1]
    outs = [jax.ShapeDtypeStruct((n, nout), jnp.float32)]
    if nwa is not None:
        outs.append(jax.ShapeDtypeStruct((n, FAT), jnp.float32))
    enc_args = () if enc is None else (enc[0][0], enc[0][1][None],
                                       enc[1][0], enc[1][1][None])
    nw_args = () if nwa is None else (nwa,)

    def body(*refs):
        c_ref, p_ref, va_ref, vb_ref, c1_ref, v2_ref, c2_ref, v3_ref, c3_ref = refs[:9]
        rest = list(refs[9:])
        if enc is not None:
            e1_ref, eb1_ref, e2_ref, eb2_ref = rest[:4]
            rest = rest[4:]
        if nwa is not None:
            nwa_ref = rest[0]
            rest = rest[1:]
        out_ref = rest[0]
        cur_v = c_ref[...]
        aggr = p_ref[...][:n]
        h = (jnp.dot(cur_v, va_ref[...], preferred_element_type=jnp.float32)
             + jnp.dot(aggr, vb_ref[...], preferred_element_type=jnp.float32)
             + c1_ref[...])
        h = jnp.maximum(h, 0.0)
        h = jnp.maximum(
            jnp.dot(h, v2_ref[...], preferred_element_type=jnp.float32) + c2_ref[...],
            0.0)
        delta = jnp.dot(h, v3_ref[...], preferred_element_type=jnp.float32) + c3_ref[...]
        if enc is None:
            residue = cur_v
        else:
            r1 = jnp.maximum(
                jnp.dot(cur_v, e1_ref[...], preferred_element_type=jnp.float32)
                + eb1_ref[...], 0.0)
            residue = jnp.maximum(
                jnp.dot(r1, e2_ref[...], preferred_element_type=jnp.float32)
                + eb2_ref[...], 0.0)
        nxt = 0.5 * residue + 0.5 * jnp.maximum(delta, 0.0)
        out_ref[...] = nxt
        if nwa is not None:
            rest[1][...] = jnp.dot(nxt, nwa_ref[...],
                                   preferred_element_type=jnp.float32)

    return pl.pallas_call(body, out_shape=outs)(
        cur, part, va, vbp, c1, v2, c2, v3, c3, *enc_args, *nw_args)


def _worker_counts(nbt):
    nbp = ((nbt + NW - 1) // NW + 7) // 8 * 8
    counts = [max(0, min(nbt - w * nbp, nbp)) for w in range(NW)]
    assert all(c % 2 == 0 and c >= 2 for c in counts)
    return nbp


@functools.lru_cache(maxsize=None)
def _sc_gather_sum_kernel(nbt, nbp, npadn):
    e = nbt * BLK
    mesh = plsc.VectorSubcoreMesh(core_axis_name="c", subcore_axis_name="s",
                                  num_cores=NC, num_subcores=NS)

    @functools.partial(
        pl.kernel, mesh=mesh,
        out_type=jax.ShapeDtypeStruct((e, TW), jnp.float32),
        scratch_types=[
            pltpu.VMEM((BLK,), jnp.int32),
            pltpu.VMEM((BLK,), jnp.int32),
            pltpu.VMEM((BLK,), jnp.int32),
            pltpu.VMEM((BLK,), jnp.int32),
            pltpu.VMEM((2, BLK, FAT), jnp.float32),
            pltpu.VMEM((2, BLK, FAT), jnp.float32),
            pltpu.VMEM((2, BLK, TW), jnp.float32),
            pltpu.SemaphoreType.DMA,
            pltpu.SemaphoreType.DMA,
        ],
    )
    def k(t_hbm, d_hbm, s_hbm, out_hbm, di0, di1, si0, si1,
          bufa, bufb, sbuf, sem0, sem1):
        sid = lax.axis_index("s")
        w = sid * NC + lax.axis_index("c")
        base = w * nbp
        nb = jnp.maximum(jnp.minimum(nbt - base, nbp), 0)

        sems = (sem0, sem1)
        dbufs = (di0, di1)
        sbufs = (si0, si1)

        def boff(kk):
            return pl.multiple_of((base + kk) * BLK, BLK)

        def start(kk, slot):
            dd, ss = dbufs[slot], sbufs[slot]
            pltpu.sync_copy(d_hbm.at[pl.ds(boff(kk), BLK)], dd)
            pltpu.sync_copy(s_hbm.at[pl.ds(boff(kk), BLK)], ss)
            pltpu.async_copy(t_hbm.at[dd], bufa.at[slot], sems[slot])
            pltpu.async_copy(t_hbm.at[ss], bufb.at[slot], sems[slot])

        def wait(slot):
            pltpu.make_async_copy(t_hbm.at[dbufs[slot]], bufa.at[slot],
                                  sems[slot]).wait()
            pltpu.make_async_copy(t_hbm.at[sbufs[slot]], bufb.at[slot],
                                  sems[slot]).wait()

        def addrows(slot):
            def row(r, _):
                for t in range(TW // 16):
                    sl = pl.ds(t * 16, 16)
                    sl2 = pl.ds(TW + t * 16, 16)
                    sbuf[slot, r, sl] = bufa[slot, r, sl] + bufb[slot, r, sl2]
                return 0
            lax.fori_loop(0, BLK, row, 0, unroll=2)

        def body(kk, _):
            pltpu.sync_copy(d_hbm.at[pl.ds(boff(kk), BLK)], di0)
            pltpu.sync_copy(s_hbm.at[pl.ds(boff(kk), BLK)], si0)
            pltpu.async_copy(t_hbm.at[di0], bufa.at[0], sem0).wait()
            pltpu.async_copy(t_hbm.at[si0], bufb.at[0], sem1).wait()
            addrows(0)
            pltpu.sync_copy(sbuf.at[0], out_hbm.at[pl.ds(boff(kk), BLK)])
            return 0

        lax.fori_loop(0, nb, body, 0)

    return k


def _sc_gather_sum(t_tab, dflat, sflat, nbt, nbp):
    k = _sc_gather_sum_kernel(nbt, nbp, t_tab.shape[0])
    return k(t_tab, dflat, sflat)


@functools.lru_cache(maxsize=None)
def _sc_scatter_add_kernel(nbt, nbtp, npad):
    half = npad // NC
    accn = -(-(half + NS) // (8 * NS)) * 8 * NS
    rows = accn // NS
    nbps = nbtp // NS
    orow = half // (NS // 2)
    mesh = plsc.VectorSubcoreMesh(core_axis_name="c", subcore_axis_name="s",
                                  num_cores=NC, num_subcores=NS)

    @functools.partial(
        pl.kernel, mesh=mesh,
        out_type=jax.ShapeDtypeStruct((npad, EOP), jnp.float32),
        scratch_types=[
            pltpu.VMEM((BLK,), jnp.int32),
            pltpu.VMEM((BLK,), jnp.int32),
            pltpu.VMEM((2, BLK, EOP), jnp.float32),
            pltpu.VMEM((rows, EOP), jnp.float32),
            pltpu.VMEM_SHARED((accn, EOP), jnp.float32),
            pltpu.SemaphoreType.DMA,
            pltpu.SemaphoreType.DMA,
        ],
    )
    def k(m_hbm, d_hbm, out_hbm, ib0, ib1, mbuf, zbuf, acc, sem0, sem1):
        cid = lax.axis_index("c")
        sid = lax.axis_index("s")
        base = sid * nbps
        nb = jnp.maximum(jnp.minimum(nbt - base, nbps), 0)

        def zrow(r, _):
            zbuf[r, pl.ds(0, EOP)] = jnp.zeros((EOP,), jnp.float32)
            return 0
        lax.fori_loop(0, rows, zrow, 0, unroll=4)
        pltpu.sync_copy(zbuf, acc.at[pl.ds(sid * rows, rows)])
        plsc.subcore_barrier()

        sems = (sem0, sem1)
        ibufs = (ib0, ib1)

        def start(kk, slot):
            off = pl.multiple_of((base + kk) * BLK, BLK)
            ioff = pl.multiple_of((cid * nbtp + base + kk) * BLK, BLK)
            pltpu.sync_copy(d_hbm.at[pl.ds(ioff, BLK)], ibufs[slot])
            pltpu.async_copy(m_hbm.at[pl.ds(off, BLK)],
                             mbuf.at[slot], sems[slot])

        def wait(slot):
            pltpu.make_async_copy(m_hbm.at[pl.ds(0, BLK)], mbuf.at[slot],
                                  sems[slot]).wait()

        start(0, 0)
        start(1, 1)

        def pair(p, _):
            for b in range(2):
                kk = 2 * p + b
                wait(b)
                pltpu.sync_copy(mbuf.at[b], acc.at[ibufs[b]], add=True)

                @pl.when(kk + 2 < nb)
                def _():
                    start(kk + 2, b)
            return 0

        lax.fori_loop(0, nb // 2, pair, 0)
        plsc.subcore_barrier()

        @pl.when(sid < NS // 2)
        def _():
            pltpu.sync_copy(acc.at[pl.ds(sid * orow, orow)],
                            out_hbm.at[pl.ds(cid * half + sid * orow, orow)])

    return k


def _sc_scatter_add(m, dscflat, nbt, nbtp, npad):
    k = _sc_scatter_add_kernel(nbt, nbtp, npad)
    return k(m, dscflat)


def kernel(x, edge_index, edge_attr, params):
    src = edge_index[0]
    dst = edge_index[1]
    e = src.shape[0]
    n = x.shape[0]
    nbt = e // BLK
    nbp = _worker_counts(nbt)
    pad_blocks = NW * nbp - nbt
    dst2 = jnp.pad(dst.reshape(nbt, BLK), ((0, pad_blocks), (0, 0)))
    src2 = jnp.pad(src.reshape(nbt, BLK), ((0, pad_blocks), (0, 0)))
    npad = -(-n // (8 * NS)) * 8 * NS

    def split_rel(rel, nin):
        (w1, b1), (w2, b2), (w3, b3) = rel
        wab = jnp.concatenate(
            [_pad_cols(w1[:nin], TW), _pad_cols(w1[nin:2 * nin], TW)], axis=1)
        wab = _pad_cols(wab, FAT)
        wc = _pad_cols(w1[2 * nin:], TW)
        b1p = _pad_cols(b1[None], TW)
        w2p = _pad_rows(w2, TW)
        w3p = _pad_cols(w3, EOP)
        b3p = _pad_cols(b3[None], EOP)
        return wab, wc, b1p, w2p, b2[None], w3p, b3p, w3.shape[1]

    def split_obj(obj, nin):
        (v1, c1), (v2, c2), (v3, c3) = obj
        va = v1[:nin]
        vbp = _pad_rows(v1[nin:], EOP)
        return va, vbp, c1[None], v2, c2[None], v3, c3[None]

    layers = params["layers"]
    encs = params["encoders"]
    nins = [x.shape[1]]
    rels = []
    for lp in layers:
        rels.append(split_rel(lp["rel"], nins[-1]))
        nins.append(lp["obj"][-1][0].shape[1])

    pad_n = lambda t: jnp.pad(t, ((0, npad - n), (0, 0)))
    dflat = dst2.reshape(-1)
    sflat = src2.reshape(-1)

    half = npad // NC
    nbtp = NW * nbp
    owner = (jnp.arange(nbtp, dtype=jnp.int32) // (nbtp // NS)) % NS
    planes = []
    for c in range(NC):
        v = dst2 - c * half
        oob = (v < 0) | (v >= half)
        planes.append(jnp.where(oob, half + owner[:, None], v))
    dscflat = jnp.concatenate(planes, axis=0).reshape(-1)
    cur = x
    ea = edge_attr
    xs = [x]
    eas = [edge_attr]
    tab = _tc_prep(x, rels[0][0])
    for li, lp in enumerate(layers):
        wab, wc, b1p, w2p, b2p, w3p, b3p, eout = rels[li]
        s = _sc_gather_sum(pad_n(tab), dflat, sflat, nbt, nbp)
        em = _tc_edge_mlp(s, ea, wc, b1p, w2p, b2p, w3p, b3p, eout)
        mp = em[0]
        m = em[1] if eout != EOP else em[0]
        part = _sc_scatter_add(mp, dscflat, nbt, nbtp, npad)
        va, vbp, c1p, v2, c2p, v3, c3p = split_obj(lp["obj"], nins[li])
        enc = encs[li]
        last = li == len(layers) - 1
        nu = _tc_node_update(
            cur, part, va, vbp, c1p, v2, c2p, v3, c3p, enc,
            None if last else rels[li + 1][0])
        cur = nu[0]
        if not last:
            tab = nu[1]
        xs.append(cur)
        eas.append(m)
        ea = m
    return (cur, xs, eas)
